# baseline (speedup 1.0000x reference)
import functools

import jax
import jax.numpy as jnp
from jax import lax
from jax.experimental import pallas as pl
from jax.experimental.pallas import tpu as pltpu
from jax.experimental.pallas import tpu_sc as plsc

B = 16384
V = 100000
D = 8
NCOLS = 37
NC, NS, L = 2, 16, 16
NW = NC * NS
BPW = B // NW
CHUNK = 128
NCHUNK = BPW // CHUNK


def _split_body(x_ref, *rest):
    o_refs = rest[:D]
    sem = rest[D]
    copies = [
        pltpu.make_async_copy(x_ref.at[d], o_refs[d], sem) for d in range(D)
    ]
    for c in copies:
        c.start()
    for c in copies:
        c.wait()


_split_tc = pl.pallas_call(
    _split_body,
    in_specs=[pl.BlockSpec(memory_space=pl.ANY)],
    out_specs=[pl.BlockSpec(memory_space=pl.ANY) for _ in range(D)],
    out_shape=[jax.ShapeDtypeStruct((V,), jnp.float32) for _ in range(D)],
    scratch_shapes=[pltpu.SemaphoreType.DMA],
)


_mesh = plsc.VectorSubcoreMesh(
    core_axis_name="c", subcore_axis_name="s", num_cores=NC, num_subcores=NS
)


@functools.partial(
    pl.kernel,
    out_type=tuple(
        jax.ShapeDtypeStruct((B,), jnp.float32) for _ in range(D)
    ),
    mesh=_mesh,
    compiler_params=pltpu.CompilerParams(
        needs_layout_passes=False, use_tc_tiling_on_sc=False
    ),
    scratch_types=[
        pltpu.VMEM((BPW,), jnp.int32),
        tuple(pltpu.VMEM((BPW,), jnp.float32) for _ in range(D)),
        pltpu.SemaphoreType.DMA,
        pltpu.SemaphoreType.DMA,
    ],
)
def _gather_sc(idx_hbm, *rest):
    tcol_hbm = rest[:D]
    out_refs = rest[D : 2 * D]
    idx_v, zd_vs, gsem, osem = rest[2 * D :]
    wid = lax.axis_index("s") * NC + lax.axis_index("c")
    base = wid * BPW

    pltpu.sync_copy(idx_hbm.at[pl.ds(base, BPW)], idx_v)
    copies = []
    for j in range(NCHUNK):
        sl = pl.ds(j * CHUNK, CHUNK)
        for d in range(D):
            copies.append(
                pltpu.async_copy(
                    tcol_hbm[d].at[idx_v.at[sl]], zd_vs[d].at[sl], gsem
                )
            )
    for c in copies:
        c.wait()

    outs = []
    for d in range(D):
        outs.append(
            pltpu.async_copy(zd_vs[d], out_refs[d].at[pl.ds(base, BPW)], osem)
        )
    for c in outs:
        c.wait()


_ASM_BLK = 512
_ASM_GRID = B // _ASM_BLK


def _assemble_body(g_ref, a_ref, o_ref, *zs_and_out):
    z_refs = zs_and_out[:D]
    out_ref = zs_and_out[D]
    out_ref[0, :] = g_ref[...].astype(jnp.float32)
    out_ref[1:8, :] = a_ref[...]
    out_ref[8:29, :] = o_ref[...]
    for d in range(D):
        out_ref[29 + d, :] = z_refs[d][...]


_assemble_tc = pl.pallas_call(
    _assemble_body,
    grid=(_ASM_GRID,),
    in_specs=(
        [pl.BlockSpec((_ASM_BLK,), lambda c: (c,))]
        + [pl.BlockSpec((7, _ASM_BLK), lambda c: (0, c))]
        + [pl.BlockSpec((21, _ASM_BLK), lambda c: (0, c))]
        + [pl.BlockSpec((_ASM_BLK,), lambda c: (c,)) for _ in range(D)]
    ),
    out_specs=pl.BlockSpec((NCOLS, _ASM_BLK), lambda c: (0, c)),
    out_shape=jax.ShapeDtypeStruct((NCOLS, B), jnp.float32),
)


def kernel(gender, age, occupation, zipcode_bucket, zipcode_table):
    tcols = _split_tc(jnp.swapaxes(zipcode_table, 0, 1))
    zcols = _gather_sc(zipcode_bucket.astype(jnp.int32), *tcols)
    out_t = _assemble_tc(
        gender.astype(jnp.int32),
        jnp.swapaxes(age, 0, 1),
        jnp.swapaxes(occupation, 0, 1),
        *zcols,
    )
    return jnp.swapaxes(out_t, 0, 1)


# speedup vs baseline: 2.0959x; 2.0959x over previous
import functools

import jax
import jax.numpy as jnp
from jax import lax
from jax.experimental import pallas as pl
from jax.experimental.pallas import tpu as pltpu
from jax.experimental.pallas import tpu_sc as plsc

B = 16384
V = 100000
D = 8
NCOLS = 37
NC, NS, L = 2, 16, 16
NW = NC * NS
BPW = B // NW
CHUNK = 128
NCHUNK = BPW // CHUNK


_TILE = 128
_VT = (V + _TILE - 1) // _TILE
_VTPAD = _VT + 1
_SPLIT_LBLK = 2048
_SPLIT_TPB = _SPLIT_LBLK // _TILE
_SPLIT_GRID = (V + _SPLIT_LBLK - 1) // _SPLIT_LBLK


def _split_body(x_ref, o_ref):
    for k in range(_SPLIT_TPB):
        o_ref[pl.ds(k * 1024, 1024)] = x_ref[
            :, pl.ds(k * _TILE, _TILE)
        ].reshape(1024)


_split_tc = pl.pallas_call(
    _split_body,
    grid=(_SPLIT_GRID,),
    in_specs=[pl.BlockSpec((D, _SPLIT_LBLK), lambda c: (0, c))],
    out_specs=pl.BlockSpec((_SPLIT_TPB * 1024,), lambda c: (c,)),
    out_shape=jax.ShapeDtypeStruct((_VTPAD * 1024,), jnp.float32),
)


_mesh = plsc.VectorSubcoreMesh(
    core_axis_name="c", subcore_axis_name="s", num_cores=NC, num_subcores=NS
)


@functools.partial(
    pl.kernel,
    out_type=tuple(
        jax.ShapeDtypeStruct((B,), jnp.float32) for _ in range(D)
    ),
    mesh=_mesh,
    compiler_params=pltpu.CompilerParams(
        needs_layout_passes=False, use_tc_tiling_on_sc=False
    ),
    scratch_types=[
        pltpu.VMEM((BPW,), jnp.int32),
        pltpu.VMEM((BPW,), jnp.int32),
        tuple(pltpu.VMEM((BPW,), jnp.float32) for _ in range(D)),
        pltpu.SemaphoreType.DMA,
        pltpu.SemaphoreType.DMA,
    ],
)
def _gather_sc(idx_hbm, tflat_hbm, *rest):
    out_refs = rest[:D]
    idx_v, addr_v, zd_vs, gsem, osem = rest[D:]
    wid = lax.axis_index("s") * NC + lax.axis_index("c")
    base = wid * BPW

    pltpu.sync_copy(idx_hbm.at[pl.ds(base, BPW)], idx_v)

    def addr_body(g, carry):
        r = idx_v[pl.ds(g * L, L)]
        addr_v[pl.ds(g * L, L)] = (
            lax.shift_left(lax.shift_right_logical(r, 7), 10)
            | lax.bitwise_and(r, 127)
        )
        return carry

    lax.fori_loop(0, BPW // L, addr_body, 0)

    copies = []
    for j in range(NCHUNK):
        sl = pl.ds(j * CHUNK, CHUNK)
        for d in range(D):
            view = tflat_hbm.at[pl.ds(d * _TILE, _VT * 1024)]
            copies.append(
                pltpu.async_copy(view.at[addr_v.at[sl]], zd_vs[d].at[sl], gsem)
            )
    for c in copies:
        c.wait()

    outs = []
    for d in range(D):
        outs.append(
            pltpu.async_copy(zd_vs[d], out_refs[d].at[pl.ds(base, BPW)], osem)
        )
    for c in outs:
        c.wait()


_ASM_BLK = 512
_ASM_GRID = B // _ASM_BLK


def _assemble_body(g_ref, a_ref, o_ref, *zs_and_out):
    z_refs = zs_and_out[:D]
    out_ref = zs_and_out[D]
    out_ref[0, :] = g_ref[...].astype(jnp.float32)
    out_ref[1:8, :] = a_ref[...]
    out_ref[8:29, :] = o_ref[...]
    for d in range(D):
        out_ref[29 + d, :] = z_refs[d][...]


_assemble_tc = pl.pallas_call(
    _assemble_body,
    grid=(_ASM_GRID,),
    in_specs=(
        [pl.BlockSpec((_ASM_BLK,), lambda c: (c,))]
        + [pl.BlockSpec((7, _ASM_BLK), lambda c: (0, c))]
        + [pl.BlockSpec((21, _ASM_BLK), lambda c: (0, c))]
        + [pl.BlockSpec((_ASM_BLK,), lambda c: (c,)) for _ in range(D)]
    ),
    out_specs=pl.BlockSpec((NCOLS, _ASM_BLK), lambda c: (0, c)),
    out_shape=jax.ShapeDtypeStruct((NCOLS, B), jnp.float32),
)


def kernel(gender, age, occupation, zipcode_bucket, zipcode_table):
    tflat = _split_tc(jnp.swapaxes(zipcode_table, 0, 1))
    zcols = _gather_sc(zipcode_bucket.astype(jnp.int32), tflat)
    out_t = _assemble_tc(
        gender.astype(jnp.int32),
        jnp.swapaxes(age, 0, 1),
        jnp.swapaxes(occupation, 0, 1),
        *zcols,
    )
    return jnp.swapaxes(out_t, 0, 1)


# speedup vs baseline: 3.6923x; 1.7617x over previous
import functools

import jax
import jax.numpy as jnp
from jax import lax
from jax.experimental import pallas as pl
from jax.experimental.pallas import tpu as pltpu
from jax.experimental.pallas import tpu_sc as plsc

B = 16384
V = 100000
D = 8
NCOLS = 37
NC, NS, L = 2, 16, 16
NW = NC * NS
BPW = B // NW
CHUNK = 128
NCHUNK = BPW // CHUNK


_TILE = 128
_VT = (V + _TILE - 1) // _TILE
_VTPAD = _VT + 1
_SPLIT_LBLK = 8192
_SPLIT_TPB = _SPLIT_LBLK // _TILE
_SPLIT_GRID = (V + _SPLIT_LBLK - 1) // _SPLIT_LBLK


def _split_body(x_ref, o_ref):
    for k in range(_SPLIT_TPB):
        o_ref[pl.ds(k * 1024, 1024)] = x_ref[
            :, pl.ds(k * _TILE, _TILE)
        ].reshape(1024)


_split_tc = pl.pallas_call(
    _split_body,
    grid=(_SPLIT_GRID,),
    in_specs=[pl.BlockSpec((D, _SPLIT_LBLK), lambda c: (0, c))],
    out_specs=pl.BlockSpec((_SPLIT_TPB * 1024,), lambda c: (c,)),
    out_shape=jax.ShapeDtypeStruct((_VTPAD * 1024,), jnp.float32),
)


_mesh = plsc.VectorSubcoreMesh(
    core_axis_name="c", subcore_axis_name="s", num_cores=NC, num_subcores=NS
)


@functools.partial(
    pl.kernel,
    out_type=tuple(
        jax.ShapeDtypeStruct((B,), jnp.float32) for _ in range(D)
    ),
    mesh=_mesh,
    compiler_params=pltpu.CompilerParams(
        needs_layout_passes=False, use_tc_tiling_on_sc=False
    ),
    scratch_types=[
        pltpu.VMEM((BPW,), jnp.int32),
        pltpu.VMEM((BPW,), jnp.int32),
        tuple(pltpu.VMEM((BPW,), jnp.float32) for _ in range(D)),
        pltpu.SemaphoreType.DMA,
        pltpu.SemaphoreType.DMA,
    ],
)
def _gather_sc(idx_hbm, tflat_hbm, *rest):
    out_refs = rest[:D]
    idx_v, addr_v, zd_vs, gsem, osem = rest[D:]
    wid = lax.axis_index("s") * NC + lax.axis_index("c")
    base = wid * BPW

    pltpu.sync_copy(idx_hbm.at[pl.ds(base, BPW)], idx_v)

    def addr_body(g, carry):
        r = idx_v[pl.ds(g * L, L)]
        addr_v[pl.ds(g * L, L)] = (
            lax.shift_left(lax.shift_right_logical(r, 7), 10)
            | lax.bitwise_and(r, 127)
        )
        return carry

    lax.fori_loop(0, BPW // L, addr_body, 0)

    copies = []
    for j in range(NCHUNK):
        sl = pl.ds(j * CHUNK, CHUNK)
        for d in range(D):
            view = tflat_hbm.at[pl.ds(d * _TILE, _VT * 1024)]
            copies.append(
                pltpu.async_copy(view.at[addr_v.at[sl]], zd_vs[d].at[sl], gsem)
            )
    for c in copies:
        c.wait()

    outs = []
    for d in range(D):
        outs.append(
            pltpu.async_copy(zd_vs[d], out_refs[d].at[pl.ds(base, BPW)], osem)
        )
    for c in outs:
        c.wait()


_ASM_BLK = 2048
_ASM_GRID = B // _ASM_BLK


def _assemble_body(g_ref, a_ref, o_ref, *zs_and_out):
    z_refs = zs_and_out[:D]
    out_ref = zs_and_out[D]
    out_ref[0, :] = g_ref[...].astype(jnp.float32)
    out_ref[1:8, :] = a_ref[...]
    out_ref[8:29, :] = o_ref[...]
    for d in range(D):
        out_ref[29 + d, :] = z_refs[d][...]


_assemble_tc = pl.pallas_call(
    _assemble_body,
    grid=(_ASM_GRID,),
    in_specs=(
        [pl.BlockSpec((_ASM_BLK,), lambda c: (c,))]
        + [pl.BlockSpec((7, _ASM_BLK), lambda c: (0, c))]
        + [pl.BlockSpec((21, _ASM_BLK), lambda c: (0, c))]
        + [pl.BlockSpec((_ASM_BLK,), lambda c: (c,)) for _ in range(D)]
    ),
    out_specs=pl.BlockSpec((NCOLS, _ASM_BLK), lambda c: (0, c)),
    out_shape=jax.ShapeDtypeStruct((NCOLS, B), jnp.float32),
)


def kernel(gender, age, occupation, zipcode_bucket, zipcode_table):
    tflat = _split_tc(jnp.swapaxes(zipcode_table, 0, 1))
    zcols = _gather_sc(zipcode_bucket.astype(jnp.int32), tflat)
    out_t = _assemble_tc(
        gender.astype(jnp.int32),
        jnp.swapaxes(age, 0, 1),
        jnp.swapaxes(occupation, 0, 1),
        *zcols,
    )
    return jnp.swapaxes(out_t, 0, 1)


# speedup vs baseline: 4.2247x; 1.1442x over previous
import functools

import jax
import jax.numpy as jnp
from jax import lax
from jax.experimental import pallas as pl
from jax.experimental.pallas import tpu as pltpu
from jax.experimental.pallas import tpu_sc as plsc

B = 16384
V = 100000
D = 8
NCOLS = 37
NC, NS, L = 2, 16, 16
NW = NC * NS
BPW = B // NW
CHUNK = 128
NCHUNK = BPW // CHUNK


_TILE = 128
_VT = (V + _TILE - 1) // _TILE
_VTPAD = _VT + 1
_SPLIT_LBLK = 16384
_SPLIT_TPB = _SPLIT_LBLK // _TILE
_SPLIT_GRID = (V + _SPLIT_LBLK - 1) // _SPLIT_LBLK


def _split_body(x_ref, o_ref):
    for k in range(_SPLIT_TPB):
        o_ref[pl.ds(k * 1024, 1024)] = x_ref[
            :, pl.ds(k * _TILE, _TILE)
        ].reshape(1024)


_split_tc = pl.pallas_call(
    _split_body,
    grid=(_SPLIT_GRID,),
    in_specs=[pl.BlockSpec((D, _SPLIT_LBLK), lambda c: (0, c))],
    out_specs=pl.BlockSpec((_SPLIT_TPB * 1024,), lambda c: (c,)),
    out_shape=jax.ShapeDtypeStruct((_VTPAD * 1024,), jnp.float32),
)


_mesh = plsc.VectorSubcoreMesh(
    core_axis_name="c", subcore_axis_name="s", num_cores=NC, num_subcores=NS
)


@functools.partial(
    pl.kernel,
    out_type=tuple(
        jax.ShapeDtypeStruct((B,), jnp.float32) for _ in range(D)
    ),
    mesh=_mesh,
    compiler_params=pltpu.CompilerParams(
        needs_layout_passes=False, use_tc_tiling_on_sc=False
    ),
    scratch_types=[
        pltpu.VMEM((BPW,), jnp.int32),
        pltpu.VMEM((BPW,), jnp.int32),
        tuple(pltpu.VMEM((BPW,), jnp.float32) for _ in range(D)),
        pltpu.SemaphoreType.DMA,
        pltpu.SemaphoreType.DMA,
    ],
)
def _gather_sc(idx_hbm, tflat_hbm, *rest):
    out_refs = rest[:D]
    idx_v, addr_v, zd_vs, gsem, osem = rest[D:]
    wid = lax.axis_index("s") * NC + lax.axis_index("c")
    base = wid * BPW

    pltpu.sync_copy(idx_hbm.at[pl.ds(base, BPW)], idx_v)

    def addr_body(g, carry):
        r = idx_v[pl.ds(g * L, L)]
        addr_v[pl.ds(g * L, L)] = (
            lax.shift_left(lax.shift_right_logical(r, 7), 10)
            | lax.bitwise_and(r, 127)
        )
        return carry

    lax.fori_loop(0, BPW // L, addr_body, 0)

    copies = []
    for j in range(NCHUNK):
        sl = pl.ds(j * CHUNK, CHUNK)
        for d in range(D):
            view = tflat_hbm.at[pl.ds(d * _TILE, _VT * 1024)]
            copies.append(
                pltpu.async_copy(view.at[addr_v.at[sl]], zd_vs[d].at[sl], gsem)
            )
    for c in copies:
        c.wait()

    outs = []
    for d in range(D):
        outs.append(
            pltpu.async_copy(zd_vs[d], out_refs[d].at[pl.ds(base, BPW)], osem)
        )
    for c in outs:
        c.wait()


_ASM_BLK = 4096
_ASM_GRID = B // _ASM_BLK


def _assemble_body(g_ref, a_ref, o_ref, *zs_and_out):
    z_refs = zs_and_out[:D]
    out_ref = zs_and_out[D]
    out_ref[0, :] = g_ref[...].astype(jnp.float32)
    out_ref[1:8, :] = a_ref[...]
    out_ref[8:29, :] = o_ref[...]
    for d in range(D):
        out_ref[29 + d, :] = z_refs[d][...]


_assemble_tc = pl.pallas_call(
    _assemble_body,
    grid=(_ASM_GRID,),
    in_specs=(
        [pl.BlockSpec((_ASM_BLK,), lambda c: (c,))]
        + [pl.BlockSpec((7, _ASM_BLK), lambda c: (0, c))]
        + [pl.BlockSpec((21, _ASM_BLK), lambda c: (0, c))]
        + [pl.BlockSpec((_ASM_BLK,), lambda c: (c,)) for _ in range(D)]
    ),
    out_specs=pl.BlockSpec((NCOLS, _ASM_BLK), lambda c: (0, c)),
    out_shape=jax.ShapeDtypeStruct((NCOLS, B), jnp.float32),
)


def kernel(gender, age, occupation, zipcode_bucket, zipcode_table):
    tflat = _split_tc(jnp.swapaxes(zipcode_table, 0, 1))
    zcols = _gather_sc(zipcode_bucket.astype(jnp.int32), tflat)
    out_t = _assemble_tc(
        gender.astype(jnp.int32),
        jnp.swapaxes(age, 0, 1),
        jnp.swapaxes(occupation, 0, 1),
        *zcols,
    )
    return jnp.swapaxes(out_t, 0, 1)


# speedup vs baseline: 4.5591x; 1.0792x over previous
import functools

import jax
import jax.numpy as jnp
from jax import lax
from jax.experimental import pallas as pl
from jax.experimental.pallas import tpu as pltpu
from jax.experimental.pallas import tpu_sc as plsc

B = 16384
V = 100000
D = 8
NCOLS = 37
NC, NS, L = 2, 16, 16
NW = NC * NS
BPW = B // NW
CHUNK = 128
NCHUNK = BPW // CHUNK


_TILE = 128
_VT = (V + _TILE - 1) // _TILE
_VTPAD = _VT + 1
_SPLIT_LBLK = 32768
_SPLIT_TPB = _SPLIT_LBLK // _TILE
_SPLIT_GRID = (V + _SPLIT_LBLK - 1) // _SPLIT_LBLK


def _split_body(x_ref, o_ref):
    for k in range(_SPLIT_TPB):
        o_ref[pl.ds(k * 1024, 1024)] = x_ref[
            :, pl.ds(k * _TILE, _TILE)
        ].reshape(1024)


_split_tc = pl.pallas_call(
    _split_body,
    grid=(_SPLIT_GRID,),
    in_specs=[pl.BlockSpec((D, _SPLIT_LBLK), lambda c: (0, c))],
    out_specs=pl.BlockSpec((_SPLIT_TPB * 1024,), lambda c: (c,)),
    out_shape=jax.ShapeDtypeStruct((_VTPAD * 1024,), jnp.float32),
)


_mesh = plsc.VectorSubcoreMesh(
    core_axis_name="c", subcore_axis_name="s", num_cores=NC, num_subcores=NS
)


@functools.partial(
    pl.kernel,
    out_type=tuple(
        jax.ShapeDtypeStruct((B,), jnp.float32) for _ in range(D)
    ),
    mesh=_mesh,
    compiler_params=pltpu.CompilerParams(
        needs_layout_passes=False, use_tc_tiling_on_sc=False
    ),
    scratch_types=[
        pltpu.VMEM((BPW,), jnp.int32),
        pltpu.VMEM((BPW,), jnp.int32),
        tuple(pltpu.VMEM((BPW,), jnp.float32) for _ in range(D)),
        pltpu.SemaphoreType.DMA,
        pltpu.SemaphoreType.DMA,
    ],
)
def _gather_sc(idx_hbm, tflat_hbm, *rest):
    out_refs = rest[:D]
    idx_v, addr_v, zd_vs, gsem, osem = rest[D:]
    wid = lax.axis_index("s") * NC + lax.axis_index("c")
    base = wid * BPW

    pltpu.sync_copy(idx_hbm.at[pl.ds(base, BPW)], idx_v)

    def addr_body(g, carry):
        r = idx_v[pl.ds(g * L, L)]
        addr_v[pl.ds(g * L, L)] = (
            lax.shift_left(lax.shift_right_logical(r, 7), 10)
            | lax.bitwise_and(r, 127)
        )
        return carry

    lax.fori_loop(0, BPW // L, addr_body, 0)

    copies = []
    for j in range(NCHUNK):
        sl = pl.ds(j * CHUNK, CHUNK)
        for d in range(D):
            view = tflat_hbm.at[pl.ds(d * _TILE, _VT * 1024)]
            copies.append(
                pltpu.async_copy(view.at[addr_v.at[sl]], zd_vs[d].at[sl], gsem)
            )
    for c in copies:
        c.wait()

    outs = []
    for d in range(D):
        outs.append(
            pltpu.async_copy(zd_vs[d], out_refs[d].at[pl.ds(base, BPW)], osem)
        )
    for c in outs:
        c.wait()


_ASM_BLK = 8192
_ASM_GRID = B // _ASM_BLK


def _assemble_body(g_ref, a_ref, o_ref, *zs_and_out):
    z_refs = zs_and_out[:D]
    out_ref = zs_and_out[D]
    out_ref[0, :] = g_ref[...].astype(jnp.float32)
    out_ref[1:8, :] = a_ref[...]
    out_ref[8:29, :] = o_ref[...]
    for d in range(D):
        out_ref[29 + d, :] = z_refs[d][...]


_assemble_tc = pl.pallas_call(
    _assemble_body,
    grid=(_ASM_GRID,),
    in_specs=(
        [pl.BlockSpec((_ASM_BLK,), lambda c: (c,))]
        + [pl.BlockSpec((7, _ASM_BLK), lambda c: (0, c))]
        + [pl.BlockSpec((21, _ASM_BLK), lambda c: (0, c))]
        + [pl.BlockSpec((_ASM_BLK,), lambda c: (c,)) for _ in range(D)]
    ),
    out_specs=pl.BlockSpec((NCOLS, _ASM_BLK), lambda c: (0, c)),
    out_shape=jax.ShapeDtypeStruct((NCOLS, B), jnp.float32),
)


def kernel(gender, age, occupation, zipcode_bucket, zipcode_table):
    tflat = _split_tc(jnp.swapaxes(zipcode_table, 0, 1))
    zcols = _gather_sc(zipcode_bucket.astype(jnp.int32), tflat)
    out_t = _assemble_tc(
        gender.astype(jnp.int32),
        jnp.swapaxes(age, 0, 1),
        jnp.swapaxes(occupation, 0, 1),
        *zcols,
    )
    return jnp.swapaxes(out_t, 0, 1)


# speedup vs baseline: 4.5719x; 1.0028x over previous
import functools

import jax
import jax.numpy as jnp
from jax import lax
from jax.experimental import pallas as pl
from jax.experimental.pallas import tpu as pltpu
from jax.experimental.pallas import tpu_sc as plsc

B = 16384
V = 100000
D = 8
NCOLS = 37
NC, NS, L = 2, 16, 16
NW = NC * NS
BPW = B // NW
CHUNK = 128
NCHUNK = BPW // CHUNK


_TILE = 128
_VT = (V + _TILE - 1) // _TILE
_VTPAD = _VT + 1
_SPLIT_LBLK = 25600
_SPLIT_TPB = _SPLIT_LBLK // _TILE
_SPLIT_GRID = (V + _SPLIT_LBLK - 1) // _SPLIT_LBLK


def _split_body(x_ref, o_ref):
    for k in range(_SPLIT_TPB):
        o_ref[pl.ds(k * 1024, 1024)] = x_ref[
            :, pl.ds(k * _TILE, _TILE)
        ].reshape(1024)


_split_tc = pl.pallas_call(
    _split_body,
    grid=(_SPLIT_GRID,),
    in_specs=[pl.BlockSpec((D, _SPLIT_LBLK), lambda c: (0, c))],
    out_specs=pl.BlockSpec((_SPLIT_TPB * 1024,), lambda c: (c,)),
    out_shape=jax.ShapeDtypeStruct((_VTPAD * 1024,), jnp.float32),
)


_mesh = plsc.VectorSubcoreMesh(
    core_axis_name="c", subcore_axis_name="s", num_cores=NC, num_subcores=NS
)


@functools.partial(
    pl.kernel,
    out_type=tuple(
        jax.ShapeDtypeStruct((B,), jnp.float32) for _ in range(D)
    ),
    mesh=_mesh,
    compiler_params=pltpu.CompilerParams(
        needs_layout_passes=False, use_tc_tiling_on_sc=False
    ),
    scratch_types=[
        pltpu.VMEM((BPW,), jnp.int32),
        pltpu.VMEM((BPW,), jnp.int32),
        tuple(pltpu.VMEM((BPW,), jnp.float32) for _ in range(D)),
        pltpu.SemaphoreType.DMA,
        pltpu.SemaphoreType.DMA,
    ],
)
def _gather_sc(idx_hbm, tflat_hbm, *rest):
    out_refs = rest[:D]
    idx_v, addr_v, zd_vs, gsem, osem = rest[D:]
    wid = lax.axis_index("s") * NC + lax.axis_index("c")
    base = wid * BPW

    pltpu.sync_copy(idx_hbm.at[pl.ds(base, BPW)], idx_v)

    def addr_body(g, carry):
        r = idx_v[pl.ds(g * L, L)]
        addr_v[pl.ds(g * L, L)] = (
            lax.shift_left(lax.shift_right_logical(r, 7), 10)
            | lax.bitwise_and(r, 127)
        )
        return carry

    lax.fori_loop(0, BPW // L, addr_body, 0)

    copies = []
    for j in range(NCHUNK):
        sl = pl.ds(j * CHUNK, CHUNK)
        for d in range(D):
            view = tflat_hbm.at[pl.ds(d * _TILE, _VT * 1024)]
            copies.append(
                pltpu.async_copy(view.at[addr_v.at[sl]], zd_vs[d].at[sl], gsem)
            )
    for c in copies:
        c.wait()

    outs = []
    for d in range(D):
        outs.append(
            pltpu.async_copy(zd_vs[d], out_refs[d].at[pl.ds(base, BPW)], osem)
        )
    for c in outs:
        c.wait()


_ASM_BLK = 8192
_ASM_GRID = B // _ASM_BLK


def _assemble_body(g_ref, a_ref, o_ref, *zs_and_out):
    z_refs = zs_and_out[:D]
    out_ref = zs_and_out[D]
    out_ref[0, :] = g_ref[...].astype(jnp.float32)
    out_ref[1:8, :] = a_ref[...]
    out_ref[8:29, :] = o_ref[...]
    for d in range(D):
        out_ref[29 + d, :] = z_refs[d][...]


_assemble_tc = pl.pallas_call(
    _assemble_body,
    grid=(_ASM_GRID,),
    in_specs=(
        [pl.BlockSpec((_ASM_BLK,), lambda c: (c,))]
        + [pl.BlockSpec((7, _ASM_BLK), lambda c: (0, c))]
        + [pl.BlockSpec((21, _ASM_BLK), lambda c: (0, c))]
        + [pl.BlockSpec((_ASM_BLK,), lambda c: (c,)) for _ in range(D)]
    ),
    out_specs=pl.BlockSpec((NCOLS, _ASM_BLK), lambda c: (0, c)),
    out_shape=jax.ShapeDtypeStruct((NCOLS, B), jnp.float32),
)


def kernel(gender, age, occupation, zipcode_bucket, zipcode_table):
    tflat = _split_tc(jnp.swapaxes(zipcode_table, 0, 1))
    zcols = _gather_sc(zipcode_bucket.astype(jnp.int32), tflat)
    out_t = _assemble_tc(
        gender.astype(jnp.int32),
        jnp.swapaxes(age, 0, 1),
        jnp.swapaxes(occupation, 0, 1),
        *zcols,
    )
    return jnp.swapaxes(out_t, 0, 1)


# speedup vs baseline: 4.6530x; 1.0177x over previous
import functools

import jax
import jax.numpy as jnp
from jax import lax
from jax.experimental import pallas as pl
from jax.experimental.pallas import tpu as pltpu
from jax.experimental.pallas import tpu_sc as plsc

B = 16384
V = 100000
D = 8
NCOLS = 37
NC, NS, L = 2, 16, 16
NW = NC * NS
BPW = B // NW
CHUNK = 128
NCHUNK = BPW // CHUNK


_TILE = 128
_VT = (V + _TILE - 1) // _TILE
_VTPAD = _VT + 1
_SPLIT_LBLK = 50048
_SPLIT_TPB = _SPLIT_LBLK // _TILE
_SPLIT_GRID = (V + _SPLIT_LBLK - 1) // _SPLIT_LBLK


def _split_body(x_ref, o_ref):
    for k in range(_SPLIT_TPB):
        o_ref[pl.ds(k * 1024, 1024)] = x_ref[
            :, pl.ds(k * _TILE, _TILE)
        ].reshape(1024)


_split_tc = pl.pallas_call(
    _split_body,
    grid=(_SPLIT_GRID,),
    in_specs=[pl.BlockSpec((D, _SPLIT_LBLK), lambda c: (0, c))],
    out_specs=pl.BlockSpec((_SPLIT_TPB * 1024,), lambda c: (c,)),
    out_shape=jax.ShapeDtypeStruct((_VTPAD * 1024,), jnp.float32),
)


_mesh = plsc.VectorSubcoreMesh(
    core_axis_name="c", subcore_axis_name="s", num_cores=NC, num_subcores=NS
)


@functools.partial(
    pl.kernel,
    out_type=tuple(
        jax.ShapeDtypeStruct((B,), jnp.float32) for _ in range(D)
    ),
    mesh=_mesh,
    compiler_params=pltpu.CompilerParams(
        needs_layout_passes=False, use_tc_tiling_on_sc=False
    ),
    scratch_types=[
        pltpu.VMEM((BPW,), jnp.int32),
        pltpu.VMEM((BPW,), jnp.int32),
        tuple(pltpu.VMEM((BPW,), jnp.float32) for _ in range(D)),
        pltpu.SemaphoreType.DMA,
        pltpu.SemaphoreType.DMA,
    ],
)
def _gather_sc(idx_hbm, tflat_hbm, *rest):
    out_refs = rest[:D]
    idx_v, addr_v, zd_vs, gsem, osem = rest[D:]
    wid = lax.axis_index("s") * NC + lax.axis_index("c")
    base = wid * BPW

    pltpu.sync_copy(idx_hbm.at[pl.ds(base, BPW)], idx_v)

    def addr_body(g, carry):
        r = idx_v[pl.ds(g * L, L)]
        addr_v[pl.ds(g * L, L)] = (
            lax.shift_left(lax.shift_right_logical(r, 7), 10)
            | lax.bitwise_and(r, 127)
        )
        return carry

    lax.fori_loop(0, BPW // L, addr_body, 0)

    copies = []
    for j in range(NCHUNK):
        sl = pl.ds(j * CHUNK, CHUNK)
        for d in range(D):
            view = tflat_hbm.at[pl.ds(d * _TILE, _VT * 1024)]
            copies.append(
                pltpu.async_copy(view.at[addr_v.at[sl]], zd_vs[d].at[sl], gsem)
            )
    for c in copies:
        c.wait()

    outs = []
    for d in range(D):
        outs.append(
            pltpu.async_copy(zd_vs[d], out_refs[d].at[pl.ds(base, BPW)], osem)
        )
    for c in outs:
        c.wait()


_ASM_BLK = 8192
_ASM_GRID = B // _ASM_BLK


def _assemble_body(g_ref, a_ref, o_ref, *zs_and_out):
    z_refs = zs_and_out[:D]
    out_ref = zs_and_out[D]
    out_ref[0, :] = g_ref[...].astype(jnp.float32)
    out_ref[1:8, :] = a_ref[...]
    out_ref[8:29, :] = o_ref[...]
    for d in range(D):
        out_ref[29 + d, :] = z_refs[d][...]


_assemble_tc = pl.pallas_call(
    _assemble_body,
    grid=(_ASM_GRID,),
    in_specs=(
        [pl.BlockSpec((_ASM_BLK,), lambda c: (c,))]
        + [pl.BlockSpec((7, _ASM_BLK), lambda c: (0, c))]
        + [pl.BlockSpec((21, _ASM_BLK), lambda c: (0, c))]
        + [pl.BlockSpec((_ASM_BLK,), lambda c: (c,)) for _ in range(D)]
    ),
    out_specs=pl.BlockSpec((NCOLS, _ASM_BLK), lambda c: (0, c)),
    out_shape=jax.ShapeDtypeStruct((NCOLS, B), jnp.float32),
)


def kernel(gender, age, occupation, zipcode_bucket, zipcode_table):
    tflat = _split_tc(jnp.swapaxes(zipcode_table, 0, 1))
    zcols = _gather_sc(zipcode_bucket.astype(jnp.int32), tflat)
    out_t = _assemble_tc(
        gender.astype(jnp.int32),
        jnp.swapaxes(age, 0, 1),
        jnp.swapaxes(occupation, 0, 1),
        *zcols,
    )
    return jnp.swapaxes(out_t, 0, 1)
